# baseline (speedup 1.0000x reference)
import jax
import jax.numpy as jnp
from jax import lax
from jax.experimental import pallas as pl
from jax.experimental.pallas import tpu as pltpu
from jax.experimental.pallas import tpu_sc as plsc

N = 10000
E = 320000
D = 128
K = 10
ALPHA = 0.1

NS = 16
L = 16

CHUNK = 128
SB = 8
GRP = SB * CHUNK
EP_TILE = -(-E // (NS * GRP)) * GRP
NG = EP_TILE // GRP
NCHUNK = EP_TILE // CHUNK
E_PAD = EP_TILE * NS

NP2 = 10240
NT = NP2 // NS
UB = 128
NUPD = NT // UB


def _body(x0_hbm, eidx, evals, zeros_hbm, xout, stg, stv, gbuf, acc, gsem, ssem, stsem):
    s = lax.axis_index("s")
    base_rows = s * NT

    def _init(u, _):
        b = base_rows + u * UB
        pltpu.sync_copy(x0_hbm.at[pl.ds(b, UB)], gbuf.at[0])
        pltpu.sync_copy(gbuf.at[0], xout.at[pl.ds(b, UB)])
        pltpu.sync_copy(zeros_hbm, acc.at[pl.ds(b, UB)])
        return 0

    lax.fori_loop(0, NUPD, _init, 0)
    plsc.subcore_barrier()

    def _step(_, carry):
        pltpu.async_copy(eidx.at[s, 0], stg.at[0], stsem)
        pltpu.async_copy(evals.at[s, 0], stv.at[0], stsem)
        pltpu.make_async_copy(eidx.at[s, 0], stg.at[0], stsem).wait()
        pltpu.make_async_copy(evals.at[s, 0], stv.at[0], stsem).wait()
        pltpu.async_copy(eidx.at[s, 1], stg.at[1], stsem)
        pltpu.async_copy(evals.at[s, 1], stv.at[1], stsem)
        pltpu.async_copy(xout.at[stg.at[0, 0, 0]], gbuf.at[0], gsem.at[0])

        def _chunk(j, _c):
            b = lax.rem(j, 2)
            g = lax.div(j, SB)
            jj = lax.rem(j, SB)
            slot = lax.rem(g, 3)

            pltpu.make_async_copy(
                xout.at[stg.at[slot, 0, jj]], gbuf.at[b], gsem.at[b]).wait()

            @pl.when(j + 1 < NCHUNK)
            def _pf():
                nb = 1 - b

                @pl.when(j >= 1)
                def _ws():
                    pltpu.make_async_copy(
                        gbuf.at[nb], acc.at[pl.ds(0, CHUNK)],
                        ssem.at[nb]).wait()

                @pl.when(jj == SB - 1)
                def _cross():
                    nslot = lax.rem(g + 1, 3)
                    pltpu.make_async_copy(
                        eidx.at[s, g + 1], stg.at[nslot], stsem).wait()
                    pltpu.make_async_copy(
                        evals.at[s, g + 1], stv.at[nslot], stsem).wait()

                    @pl.when(g + 2 < NG)
                    def _st():
                        pltpu.async_copy(
                            eidx.at[s, g + 2],
                            stg.at[lax.rem(g + 2, 3)], stsem)
                        pltpu.async_copy(
                            evals.at[s, g + 2],
                            stv.at[lax.rem(g + 2, 3)], stsem)

                g1 = lax.div(j + 1, SB)
                jj1 = lax.rem(j + 1, SB)
                slot1 = lax.rem(g1, 3)
                pltpu.async_copy(
                    xout.at[stg.at[slot1, 0, jj1]], gbuf.at[nb], gsem.at[nb])

            def _scale(q, _e):
                vv = stv[slot, jj, pl.ds(q * L, L)]
                for i in range(L):
                    v = vv[jnp.full((L,), i, jnp.int32)]
                    e = q * L + i
                    for f in range(D // L):
                        sl = pl.ds(f * L, L)
                        gbuf[b, e, sl] = gbuf[b, e, sl] * v
                return 0

            lax.fori_loop(0, CHUNK // L, _scale, 0)

            pltpu.async_copy(
                gbuf.at[b], acc.at[stg.at[slot, 1, jj]], ssem.at[b],
                add=True)
            return 0

        lax.fori_loop(0, NCHUNK, _chunk, 0)

        pltpu.make_async_copy(
            gbuf.at[0], acc.at[pl.ds(0, CHUNK)], ssem.at[0]).wait()
        pltpu.make_async_copy(
            gbuf.at[1], acc.at[pl.ds(0, CHUNK)], ssem.at[1]).wait()
        plsc.subcore_barrier()

        def _upd(u, _u):
            b = base_rows + u * UB
            pltpu.sync_copy(acc.at[pl.ds(b, UB)], gbuf.at[0])
            pltpu.sync_copy(x0_hbm.at[pl.ds(b, UB)], gbuf.at[1])

            def _mix(i, _i):
                for f in range(D // L):
                    sl = pl.ds(f * L, L)
                    gbuf[0, i, sl] = (1.0 - ALPHA) * gbuf[0, i, sl] \
                        + ALPHA * gbuf[1, i, sl]
                return 0

            lax.fori_loop(0, UB, _mix, 0)
            pltpu.sync_copy(gbuf.at[0], xout.at[pl.ds(b, UB)])
            pltpu.sync_copy(zeros_hbm, acc.at[pl.ds(b, UB)])
            return 0

        lax.fori_loop(0, NUPD, _upd, 0)
        plsc.subcore_barrier()
        return carry

    lax.fori_loop(0, K, _step, 0)


@jax.jit
def kernel(x, adj_indices, adj_values):
    row = adj_indices[0].astype(jnp.int32)
    col = adj_indices[1].astype(jnp.int32)
    val = adj_values.astype(jnp.float32)

    pad = E_PAD - E
    row = jnp.concatenate([row, jnp.zeros((pad,), jnp.int32)])
    col = jnp.concatenate([col, jnp.zeros((pad,), jnp.int32)])
    val = jnp.concatenate([val, jnp.zeros((pad,), jnp.float32)])

    eidx = jnp.stack([
        col.reshape(NS, NG, SB, CHUNK),
        row.reshape(NS, NG, SB, CHUNK),
    ], axis=2)
    evals = val.reshape(NS, NG, SB, CHUNK)

    x0 = jnp.pad(x, ((0, NP2 - N), (0, 0)))
    zeros = jnp.zeros((UB, D), jnp.float32)

    mesh = plsc.VectorSubcoreMesh(
        core_axis_name="c", subcore_axis_name="s", num_cores=1)
    xout = pl.kernel(
        _body,
        out_type=jax.ShapeDtypeStruct((NP2, D), jnp.float32),
        mesh=mesh,
        scratch_types=[
            pltpu.VMEM((3, 2, SB, CHUNK), jnp.int32),
            pltpu.VMEM((3, SB, CHUNK), jnp.float32),
            pltpu.VMEM((2, CHUNK, D), jnp.float32),
            pltpu.VMEM_SHARED((NP2, D), jnp.float32),
            pltpu.SemaphoreType.DMA((2,)),
            pltpu.SemaphoreType.DMA((2,)),
            pltpu.SemaphoreType.DMA,
        ],
    )(x0, eidx, evals, zeros)

    return xout[:N]


# speedup vs baseline: 1.1037x; 1.1037x over previous
import jax
import jax.numpy as jnp
from jax import lax
from jax.experimental import pallas as pl
from jax.experimental.pallas import tpu as pltpu
from jax.experimental.pallas import tpu_sc as plsc

N = 10000
E = 320000
D = 128
K = 10
ALPHA = 0.1

NS = 16
L = 16

CHUNK = 64
NB = 4
LA = NB - 1
SB = 8
GRP = SB * CHUNK
EP_TILE = -(-E // (NS * GRP)) * GRP
NG = EP_TILE // GRP
NCHUNK = EP_TILE // CHUNK
E_PAD = EP_TILE * NS

NP2 = 10240
NT = NP2 // NS
UB = 64
NUPD = NT // UB


def _body(x0_hbm, eidx, evals, zeros_hbm, xout,
          stg, stv, gbuf, acc, gsem, ssem, stsem):
    s = lax.axis_index("s")
    base_rows = s * NT

    def _init(u, _):
        b = base_rows + u * UB
        pltpu.sync_copy(x0_hbm.at[pl.ds(b, UB)], gbuf.at[0])
        pltpu.sync_copy(gbuf.at[0], xout.at[pl.ds(b, UB)])
        pltpu.sync_copy(zeros_hbm, acc.at[pl.ds(b, UB)])
        return 0

    lax.fori_loop(0, NUPD, _init, 0)
    plsc.subcore_barrier()

    def _step(_, carry):
        pltpu.async_copy(eidx.at[s, 0], stg.at[0], stsem)
        pltpu.async_copy(evals.at[s, 0], stv.at[0], stsem)
        pltpu.make_async_copy(eidx.at[s, 0], stg.at[0], stsem).wait()
        pltpu.make_async_copy(evals.at[s, 0], stv.at[0], stsem).wait()
        pltpu.async_copy(eidx.at[s, 1], stg.at[1], stsem)
        pltpu.async_copy(evals.at[s, 1], stv.at[1], stsem)
        for p in range(LA):
            pltpu.async_copy(
                xout.at[stg.at[0, 0, p]], gbuf.at[p], gsem.at[p])

        def _chunk(j, _c):
            b = lax.rem(j, NB)
            g = lax.div(j, SB)
            jj = lax.rem(j, SB)
            slot = lax.rem(g, 3)

            pltpu.make_async_copy(
                xout.at[stg.at[slot, 0, jj]], gbuf.at[b], gsem.at[b]).wait()

            @pl.when(j + LA < NCHUNK)
            def _pf():
                nb = lax.rem(j + LA, NB)

                @pl.when(j >= 1)
                def _ws():
                    pltpu.make_async_copy(
                        gbuf.at[nb], acc.at[pl.ds(0, CHUNK)],
                        ssem.at[nb]).wait()

                @pl.when(lax.rem(j + LA, SB) == 0)
                def _cross():
                    gx = lax.div(j + LA, SB)
                    nslot = lax.rem(gx, 3)
                    pltpu.make_async_copy(
                        eidx.at[s, gx], stg.at[nslot], stsem).wait()
                    pltpu.make_async_copy(
                        evals.at[s, gx], stv.at[nslot], stsem).wait()

                    @pl.when(gx + 1 < NG)
                    def _st():
                        pltpu.async_copy(
                            eidx.at[s, gx + 1],
                            stg.at[lax.rem(gx + 1, 3)], stsem)
                        pltpu.async_copy(
                            evals.at[s, gx + 1],
                            stv.at[lax.rem(gx + 1, 3)], stsem)

                g1 = lax.div(j + LA, SB)
                jj1 = lax.rem(j + LA, SB)
                slot1 = lax.rem(g1, 3)
                pltpu.async_copy(
                    xout.at[stg.at[slot1, 0, jj1]], gbuf.at[nb], gsem.at[nb])

            def _scale(q, _e):
                vv = stv[slot, jj, pl.ds(q * L, L)]
                for i in range(L):
                    v = vv[jnp.full((L,), i, jnp.int32)]
                    e = q * L + i
                    for f in range(D // L):
                        sl = pl.ds(f * L, L)
                        gbuf[b, e, sl] = gbuf[b, e, sl] * v
                return 0

            lax.fori_loop(0, CHUNK // L, _scale, 0)

            pltpu.async_copy(
                gbuf.at[b], acc.at[stg.at[slot, 1, jj]], ssem.at[b],
                add=True)
            return 0

        lax.fori_loop(0, NCHUNK, _chunk, 0)

        for p in range(NB):
            pltpu.make_async_copy(
                gbuf.at[p], acc.at[pl.ds(0, CHUNK)], ssem.at[p]).wait()
        plsc.subcore_barrier()

        def _upd(u, _u):
            b = base_rows + u * UB
            pltpu.sync_copy(acc.at[pl.ds(b, UB)], gbuf.at[0])
            pltpu.sync_copy(x0_hbm.at[pl.ds(b, UB)], gbuf.at[1])

            def _mix(i, _i):
                for f in range(D // L):
                    sl = pl.ds(f * L, L)
                    gbuf[0, i, sl] = (1.0 - ALPHA) * gbuf[0, i, sl] \
                        + ALPHA * gbuf[1, i, sl]
                return 0

            lax.fori_loop(0, UB, _mix, 0)
            pltpu.sync_copy(gbuf.at[0], xout.at[pl.ds(b, UB)])
            pltpu.sync_copy(zeros_hbm, acc.at[pl.ds(b, UB)])
            return 0

        lax.fori_loop(0, NUPD, _upd, 0)
        plsc.subcore_barrier()
        return carry

    lax.fori_loop(0, K, _step, 0)


@jax.jit
def kernel(x, adj_indices, adj_values):
    row = adj_indices[0].astype(jnp.int32)
    col = adj_indices[1].astype(jnp.int32)
    val = adj_values.astype(jnp.float32)

    pad = E_PAD - E
    row = jnp.concatenate([row, jnp.zeros((pad,), jnp.int32)])
    col = jnp.concatenate([col, jnp.zeros((pad,), jnp.int32)])
    val = jnp.concatenate([val, jnp.zeros((pad,), jnp.float32)])

    eidx = jnp.stack([
        col.reshape(NS, NG, SB, CHUNK),
        row.reshape(NS, NG, SB, CHUNK),
    ], axis=2)
    evals = val.reshape(NS, NG, SB, CHUNK)

    x0 = jnp.pad(x, ((0, NP2 - N), (0, 0)))
    zeros = jnp.zeros((UB, D), jnp.float32)

    mesh = plsc.VectorSubcoreMesh(
        core_axis_name="c", subcore_axis_name="s", num_cores=1)
    xout = pl.kernel(
        _body,
        out_type=jax.ShapeDtypeStruct((NP2, D), jnp.float32),
        mesh=mesh,
        scratch_types=[
            pltpu.VMEM((3, 2, SB, CHUNK), jnp.int32),
            pltpu.VMEM((3, SB, CHUNK), jnp.float32),
            pltpu.VMEM((NB, CHUNK, D), jnp.float32),
            pltpu.VMEM_SHARED((NP2, D), jnp.float32),
            pltpu.SemaphoreType.DMA((NB,)),
            pltpu.SemaphoreType.DMA((NB,)),
            pltpu.SemaphoreType.DMA,
        ],
    )(x0, eidx, evals, zeros)

    return xout[:N]


# speedup vs baseline: 1.5186x; 1.3759x over previous
import functools

import jax
import jax.numpy as jnp
from jax import lax
from jax.experimental import pallas as pl
from jax.experimental.pallas import tpu as pltpu
from jax.experimental.pallas import tpu_sc as plsc

N = 10000
E = 320000
D = 128
K = 10
ALPHA = 0.1

NC = 2
NS = 16
L = 16

CHUNK = 64
NB = 4
LA = NB - 1
SB = 8
GRP = SB * CHUNK
EP_TILE = -(-E // (NC * NS * GRP)) * GRP
NG = EP_TILE // GRP
NCHUNK = EP_TILE // CHUNK
E_PAD = EP_TILE * NC * NS

NP2 = 10240
NT = NP2 // NS
UB = 64
NUPD = NT // UB
NTF = NP2 // (NC * NS)
NUPF = NTF // UB


def _phase_b1_dump(c, s, xc, eidx, evals, acc,
                   stg, stv, gbuf, gsem, ssem, stsem, partials):
    base_rows = s * NT
    cbase = c * NP2

    pltpu.async_copy(eidx.at[c, s, 0], stg.at[0], stsem)
    pltpu.async_copy(evals.at[c, s, 0], stv.at[0], stsem)
    pltpu.make_async_copy(eidx.at[c, s, 0], stg.at[0], stsem).wait()
    pltpu.make_async_copy(evals.at[c, s, 0], stv.at[0], stsem).wait()
    pltpu.async_copy(eidx.at[c, s, 1], stg.at[1], stsem)
    pltpu.async_copy(evals.at[c, s, 1], stv.at[1], stsem)
    for p in range(LA):
        g0, j0 = p // SB, p % SB
        pltpu.async_copy(
            xc.at[stg.at[g0, 0, j0]], gbuf.at[p], gsem.at[p])

    def _chunk(j, _c):
        b = lax.rem(j, NB)
        g = lax.div(j, SB)
        jj = lax.rem(j, SB)
        slot = lax.rem(g, 3)

        pltpu.make_async_copy(
            xc.at[stg.at[slot, 0, jj]], gbuf.at[b], gsem.at[b]).wait()

        @pl.when(j + LA < NCHUNK)
        def _pf():
            nb = lax.rem(j + LA, NB)

            @pl.when(j >= 1)
            def _ws():
                pltpu.make_async_copy(
                    gbuf.at[nb], acc.at[pl.ds(0, CHUNK)],
                    ssem.at[nb]).wait()

            @pl.when(lax.rem(j + LA, SB) == 0)
            def _cross():
                gx = lax.div(j + LA, SB)
                nslot = lax.rem(gx, 3)
                pltpu.make_async_copy(
                    eidx.at[c, s, gx], stg.at[nslot], stsem).wait()
                pltpu.make_async_copy(
                    evals.at[c, s, gx], stv.at[nslot], stsem).wait()

                @pl.when(gx + 1 < NG)
                def _st():
                    pltpu.async_copy(
                        eidx.at[c, s, gx + 1],
                        stg.at[lax.rem(gx + 1, 3)], stsem)
                    pltpu.async_copy(
                        evals.at[c, s, gx + 1],
                        stv.at[lax.rem(gx + 1, 3)], stsem)

            g1 = lax.div(j + LA, SB)
            jj1 = lax.rem(j + LA, SB)
            slot1 = lax.rem(g1, 3)
            pltpu.async_copy(
                xc.at[stg.at[slot1, 0, jj1]], gbuf.at[nb], gsem.at[nb])

        def _scale(q, _e):
            vv = stv[slot, jj, pl.ds(q * L, L)]
            for i in range(L):
                v = vv[jnp.full((L,), i, jnp.int32)]
                e = q * L + i
                for f in range(D // L):
                    sl = pl.ds(f * L, L)
                    gbuf[b, e, sl] = gbuf[b, e, sl] * v
            return 0

        lax.fori_loop(0, CHUNK // L, _scale, 0)

        pltpu.async_copy(
            gbuf.at[b], acc.at[stg.at[slot, 1, jj]], ssem.at[b],
            add=True)
        return 0

    lax.fori_loop(0, NCHUNK, _chunk, 0)

    for p in range(NB):
        pltpu.make_async_copy(
            gbuf.at[p], acc.at[pl.ds(0, CHUNK)], ssem.at[p]).wait()
    plsc.subcore_barrier()

    def _dump(u, _):
        b = base_rows + u * UB
        pltpu.sync_copy(
            acc.at[pl.ds(b, UB)], partials.at[pl.ds(cbase + b, UB)])
        return 0

    lax.fori_loop(0, NUPD, _dump, 0)


def _mix_rows(dst, a_ref, h_ref):
    def _mix(i, _):
        for f in range(D // L):
            sl = pl.ds(f * L, L)
            dst[i, sl] = (1.0 - ALPHA) * (dst[i, sl] + a_ref[i, sl]) \
                + ALPHA * h_ref[i, sl]
        return 0

    lax.fori_loop(0, UB, _mix, 0)


def _body_first(x0_hbm, eidx, evals, zeros_hbm, xc, partials,
                stg, stv, gbuf, acc, gsem, ssem, stsem):
    c = lax.axis_index("c")
    s = lax.axis_index("s")
    base_rows = s * NT
    cbase = c * NP2

    def _init(u, _):
        b = base_rows + u * UB
        pltpu.sync_copy(x0_hbm.at[pl.ds(b, UB)], gbuf.at[0])
        pltpu.sync_copy(gbuf.at[0], xc.at[pl.ds(cbase + b, UB)])
        pltpu.sync_copy(zeros_hbm, acc.at[pl.ds(b, UB)])
        return 0

    lax.fori_loop(0, NUPD, _init, 0)
    plsc.subcore_barrier()
    _phase_b1_dump(c, s, xc, eidx, evals, acc,
                   stg, stv, gbuf, gsem, ssem, stsem, partials)


def _body_mid(p01, x0_hbm, eidx, evals, zeros_hbm, xc, partials,
              stg, stv, gbuf, acc, gsem, ssem, stsem):
    c = lax.axis_index("c")
    s = lax.axis_index("s")
    base_rows = s * NT
    cbase = c * NP2

    def _upd(u, _):
        b = base_rows + u * UB
        pltpu.sync_copy(p01.at[pl.ds(b, UB)], gbuf.at[0])
        pltpu.sync_copy(p01.at[pl.ds(NP2 + b, UB)], gbuf.at[1])
        pltpu.sync_copy(x0_hbm.at[pl.ds(b, UB)], gbuf.at[2])
        _mix_rows(gbuf.at[0], gbuf.at[1], gbuf.at[2])
        pltpu.sync_copy(gbuf.at[0], xc.at[pl.ds(cbase + b, UB)])
        pltpu.sync_copy(zeros_hbm, acc.at[pl.ds(b, UB)])
        return 0

    lax.fori_loop(0, NUPD, _upd, 0)
    plsc.subcore_barrier()
    _phase_b1_dump(c, s, xc, eidx, evals, acc,
                   stg, stv, gbuf, gsem, ssem, stsem, partials)


def _body_last(p01, x0_hbm, xfin, gbuf):
    c = lax.axis_index("c")
    s = lax.axis_index("s")
    base_rows = (c * NS + s) * NTF

    def _upd(u, _):
        b = base_rows + u * UB
        pltpu.sync_copy(p01.at[pl.ds(b, UB)], gbuf.at[0])
        pltpu.sync_copy(p01.at[pl.ds(NP2 + b, UB)], gbuf.at[1])
        pltpu.sync_copy(x0_hbm.at[pl.ds(b, UB)], gbuf.at[2])
        _mix_rows(gbuf.at[0], gbuf.at[1], gbuf.at[2])
        pltpu.sync_copy(gbuf.at[0], xfin.at[pl.ds(b, UB)])
        return 0

    lax.fori_loop(0, NUPF, _upd, 0)


@jax.jit
def kernel(x, adj_indices, adj_values):
    row = adj_indices[0].astype(jnp.int32)
    col = adj_indices[1].astype(jnp.int32)
    val = adj_values.astype(jnp.float32)

    pad = E_PAD - E
    row = jnp.concatenate([row, jnp.zeros((pad,), jnp.int32)])
    col = jnp.concatenate([col, jnp.zeros((pad,), jnp.int32)])
    val = jnp.concatenate([val, jnp.zeros((pad,), jnp.float32)])

    colc = col.reshape(NC, NS, NG, SB, CHUNK)
    colc = colc + (jnp.arange(NC, dtype=jnp.int32) * NP2).reshape(
        NC, 1, 1, 1, 1)
    eidx = jnp.stack([
        colc,
        row.reshape(NC, NS, NG, SB, CHUNK),
    ], axis=3)
    evals = val.reshape(NC, NS, NG, SB, CHUNK)

    x0 = jnp.pad(x, ((0, NP2 - N), (0, 0)))
    zeros = jnp.zeros((UB, D), jnp.float32)

    mesh = plsc.VectorSubcoreMesh(
        core_axis_name="c", subcore_axis_name="s", num_cores=NC)
    scratch = [
        pltpu.VMEM((3, 2, SB, CHUNK), jnp.int32),
        pltpu.VMEM((3, SB, CHUNK), jnp.float32),
        pltpu.VMEM((NB, CHUNK, D), jnp.float32),
        pltpu.VMEM_SHARED((NP2, D), jnp.float32),
        pltpu.SemaphoreType.DMA((NB,)),
        pltpu.SemaphoreType.DMA((NB,)),
        pltpu.SemaphoreType.DMA,
    ]
    state_t = [
        jax.ShapeDtypeStruct((NC * NP2, D), jnp.float32),
        jax.ShapeDtypeStruct((NC * NP2, D), jnp.float32),
    ]

    first = pl.kernel(_body_first, out_type=state_t, mesh=mesh,
                      scratch_types=scratch)
    mid = pl.kernel(_body_mid, out_type=state_t, mesh=mesh,
                    scratch_types=scratch)
    last = pl.kernel(
        _body_last,
        out_type=jax.ShapeDtypeStruct((NP2, D), jnp.float32),
        mesh=mesh,
        scratch_types=[pltpu.VMEM((3, UB, D), jnp.float32)])

    _, parts = first(x0, eidx, evals, zeros)
    for _ in range(K - 1):
        _, parts = mid(parts, x0, eidx, evals, zeros)
    xfin = last(parts, x0)
    return xfin[:N]


# speedup vs baseline: 1.8939x; 1.2471x over previous
import functools

import jax
import jax.numpy as jnp
from jax import lax
from jax.experimental import pallas as pl
from jax.experimental.pallas import tpu as pltpu
from jax.experimental.pallas import tpu_sc as plsc

N = 10000
E = 320000
D = 128
K = 10
ALPHA = 0.1

NC = 2
NS = 16
L = 16

CHUNK = 64
NB = 4
LA = NB - 1
SB = 8
GRP = SB * CHUNK
EP_TILE = -(-E // (NC * NS * GRP)) * GRP
NG = EP_TILE // GRP
NCHUNK = EP_TILE // CHUNK
E_PAD = EP_TILE * NC * NS

NP2 = 10240
NT = NP2 // NS
UB = 64
NUPD = NT // UB
NTF = NP2 // (NC * NS)
NUPF = NTF // UB


def _body_b1(x, eidx, evals, zeros_hbm, partials,
             stg, stv, gbuf, acc, gsem, ssem, stsem):
    c = lax.axis_index("c")
    s = lax.axis_index("s")
    base_rows = s * NT
    cbase = c * NP2

    pltpu.sync_copy(zeros_hbm, acc.at[pl.ds(base_rows, NT)])
    plsc.subcore_barrier()

    pltpu.async_copy(eidx.at[c, s, 0], stg.at[0], stsem)
    pltpu.async_copy(evals.at[c, s, 0], stv.at[0], stsem)
    pltpu.make_async_copy(eidx.at[c, s, 0], stg.at[0], stsem).wait()
    pltpu.make_async_copy(evals.at[c, s, 0], stv.at[0], stsem).wait()
    pltpu.async_copy(eidx.at[c, s, 1], stg.at[1], stsem)
    pltpu.async_copy(evals.at[c, s, 1], stv.at[1], stsem)
    for p in range(LA):
        g0, j0 = p // SB, p % SB
        pltpu.async_copy(
            x.at[stg.at[g0, 0, j0]], gbuf.at[p], gsem.at[p])

    def _chunk(j, _c):
        b = lax.rem(j, NB)
        g = lax.div(j, SB)
        jj = lax.rem(j, SB)
        slot = lax.rem(g, 3)

        pltpu.make_async_copy(
            x.at[stg.at[slot, 0, jj]], gbuf.at[b], gsem.at[b]).wait()

        @pl.when(j + LA < NCHUNK)
        def _pf():
            nb = lax.rem(j + LA, NB)

            @pl.when(j >= 1)
            def _ws():
                pltpu.make_async_copy(
                    gbuf.at[nb], acc.at[pl.ds(0, CHUNK)],
                    ssem.at[nb]).wait()

            @pl.when(lax.rem(j + LA, SB) == 0)
            def _cross():
                gx = lax.div(j + LA, SB)
                nslot = lax.rem(gx, 3)
                pltpu.make_async_copy(
                    eidx.at[c, s, gx], stg.at[nslot], stsem).wait()
                pltpu.make_async_copy(
                    evals.at[c, s, gx], stv.at[nslot], stsem).wait()

                @pl.when(gx + 1 < NG)
                def _st():
                    pltpu.async_copy(
                        eidx.at[c, s, gx + 1],
                        stg.at[lax.rem(gx + 1, 3)], stsem)
                    pltpu.async_copy(
                        evals.at[c, s, gx + 1],
                        stv.at[lax.rem(gx + 1, 3)], stsem)

            g1 = lax.div(j + LA, SB)
            jj1 = lax.rem(j + LA, SB)
            slot1 = lax.rem(g1, 3)
            pltpu.async_copy(
                x.at[stg.at[slot1, 0, jj1]], gbuf.at[nb], gsem.at[nb])

        def _scale(q, _e):
            vv = stv[slot, jj, pl.ds(q * L, L)]
            for i in range(L):
                v = vv[jnp.full((L,), i, jnp.int32)]
                e = q * L + i
                for f in range(D // L):
                    sl = pl.ds(f * L, L)
                    gbuf[b, e, sl] = gbuf[b, e, sl] * v
            return 0

        lax.fori_loop(0, CHUNK // L, _scale, 0)

        pltpu.async_copy(
            gbuf.at[b], acc.at[stg.at[slot, 1, jj]], ssem.at[b],
            add=True)
        return 0

    lax.fori_loop(0, NCHUNK, _chunk, 0)

    for p in range(NB):
        pltpu.make_async_copy(
            gbuf.at[p], acc.at[pl.ds(0, CHUNK)], ssem.at[p]).wait()
    plsc.subcore_barrier()

    def _dump(u, _):
        b = base_rows + u * UB
        pltpu.sync_copy(
            acc.at[pl.ds(b, UB)], partials.at[pl.ds(cbase + b, UB)])
        return 0

    lax.fori_loop(0, NUPD, _dump, 0)


def _mix_rows(dst, a_ref, h_ref):
    def _mix(i, _):
        for f in range(D // L):
            sl = pl.ds(f * L, L)
            dst[i, sl] = (1.0 - ALPHA) * (dst[i, sl] + a_ref[i, sl]) \
                + ALPHA * h_ref[i, sl]
        return 0

    lax.fori_loop(0, UB, _mix, 0)


def _body_upd(p01, x0_hbm, xfin, gbuf):
    c = lax.axis_index("c")
    s = lax.axis_index("s")
    base_rows = (c * NS + s) * NTF

    def _upd(u, _):
        b = base_rows + u * UB
        pltpu.sync_copy(p01.at[pl.ds(b, UB)], gbuf.at[0])
        pltpu.sync_copy(p01.at[pl.ds(NP2 + b, UB)], gbuf.at[1])
        pltpu.sync_copy(x0_hbm.at[pl.ds(b, UB)], gbuf.at[2])
        _mix_rows(gbuf.at[0], gbuf.at[1], gbuf.at[2])
        pltpu.sync_copy(gbuf.at[0], xfin.at[pl.ds(b, UB)])
        return 0

    lax.fori_loop(0, NUPF, _upd, 0)


@jax.jit
def kernel(x, adj_indices, adj_values):
    row = adj_indices[0].astype(jnp.int32)
    col = adj_indices[1].astype(jnp.int32)
    val = adj_values.astype(jnp.float32)

    pad = E_PAD - E
    row = jnp.concatenate([row, jnp.zeros((pad,), jnp.int32)])
    col = jnp.concatenate([col, jnp.zeros((pad,), jnp.int32)])
    val = jnp.concatenate([val, jnp.zeros((pad,), jnp.float32)])

    eidx = jnp.stack([
        col.reshape(NC, NS, NG, SB, CHUNK),
        row.reshape(NC, NS, NG, SB, CHUNK),
    ], axis=3)
    evals = val.reshape(NC, NS, NG, SB, CHUNK)

    x0 = jnp.pad(x, ((0, NP2 - N), (0, 0)))
    zeros = jnp.zeros((NT, D), jnp.float32)

    mesh = plsc.VectorSubcoreMesh(
        core_axis_name="c", subcore_axis_name="s", num_cores=NC)
    scratch = [
        pltpu.VMEM((3, 2, SB, CHUNK), jnp.int32),
        pltpu.VMEM((3, SB, CHUNK), jnp.float32),
        pltpu.VMEM((NB, CHUNK, D), jnp.float32),
        pltpu.VMEM_SHARED((NP2, D), jnp.float32),
        pltpu.SemaphoreType.DMA((NB,)),
        pltpu.SemaphoreType.DMA((NB,)),
        pltpu.SemaphoreType.DMA,
    ]

    b1 = pl.kernel(
        _body_b1,
        out_type=jax.ShapeDtypeStruct((NC * NP2, D), jnp.float32),
        mesh=mesh, scratch_types=scratch)
    upd = pl.kernel(
        _body_upd,
        out_type=jax.ShapeDtypeStruct((NP2, D), jnp.float32),
        mesh=mesh,
        scratch_types=[pltpu.VMEM((3, UB, D), jnp.float32)])

    parts = b1(x0, eidx, evals, zeros)
    for _ in range(K - 1):
        parts = b1(upd(parts, x0), eidx, evals, zeros)
    xfin = upd(parts, x0)
    return xfin[:N]


# speedup vs baseline: 2.0264x; 1.0700x over previous
import functools

import jax
import jax.numpy as jnp
from jax import lax
from jax.experimental import pallas as pl
from jax.experimental.pallas import tpu as pltpu
from jax.experimental.pallas import tpu_sc as plsc

N = 10000
E = 320000
D = 128
K = 10
ALPHA = 0.1

NC = 2
NS = 16
L = 16

CHUNK = 80
NB = 4
LA = NB - 1
SB = 4
GRP = SB * CHUNK
EP_TILE = -(-E // (NC * NS * GRP)) * GRP
NG = EP_TILE // GRP
NCHUNK = EP_TILE // CHUNK
E_PAD = EP_TILE * NC * NS

NP2 = 10240
NT = NP2 // NS
UB = 64
NUPD = NT // UB
NTF = NP2 // (NC * NS)
NUPF = NTF // UB


def _body_b1(x, eidx, evals, zeros_hbm, partials,
             stg, stv, gbuf, acc, gsem, ssem, stsem):
    c = lax.axis_index("c")
    s = lax.axis_index("s")
    base_rows = s * NT
    cbase = c * NP2

    pltpu.sync_copy(zeros_hbm, acc.at[pl.ds(base_rows, NT)])
    plsc.subcore_barrier()

    pltpu.async_copy(eidx.at[c, s, 0], stg.at[0], stsem)
    pltpu.async_copy(evals.at[c, s, 0], stv.at[0], stsem)
    pltpu.make_async_copy(eidx.at[c, s, 0], stg.at[0], stsem).wait()
    pltpu.make_async_copy(evals.at[c, s, 0], stv.at[0], stsem).wait()
    pltpu.async_copy(eidx.at[c, s, 1], stg.at[1], stsem)
    pltpu.async_copy(evals.at[c, s, 1], stv.at[1], stsem)
    for p in range(LA):
        g0, j0 = p // SB, p % SB
        pltpu.async_copy(
            x.at[stg.at[g0, 0, j0]], gbuf.at[p], gsem.at[p])

    def _chunk(j, _c):
        b = lax.rem(j, NB)
        g = lax.div(j, SB)
        jj = lax.rem(j, SB)
        slot = lax.rem(g, 3)

        pltpu.make_async_copy(
            x.at[stg.at[slot, 0, jj]], gbuf.at[b], gsem.at[b]).wait()

        @pl.when(j + LA < NCHUNK)
        def _pf():
            nb = lax.rem(j + LA, NB)

            @pl.when(j >= 1)
            def _ws():
                pltpu.make_async_copy(
                    gbuf.at[nb], acc.at[pl.ds(0, CHUNK)],
                    ssem.at[nb]).wait()

            @pl.when(lax.rem(j + LA, SB) == 0)
            def _cross():
                gx = lax.div(j + LA, SB)
                nslot = lax.rem(gx, 3)
                pltpu.make_async_copy(
                    eidx.at[c, s, gx], stg.at[nslot], stsem).wait()
                pltpu.make_async_copy(
                    evals.at[c, s, gx], stv.at[nslot], stsem).wait()

                @pl.when(gx + 1 < NG)
                def _st():
                    pltpu.async_copy(
                        eidx.at[c, s, gx + 1],
                        stg.at[lax.rem(gx + 1, 3)], stsem)
                    pltpu.async_copy(
                        evals.at[c, s, gx + 1],
                        stv.at[lax.rem(gx + 1, 3)], stsem)

            g1 = lax.div(j + LA, SB)
            jj1 = lax.rem(j + LA, SB)
            slot1 = lax.rem(g1, 3)
            pltpu.async_copy(
                x.at[stg.at[slot1, 0, jj1]], gbuf.at[nb], gsem.at[nb])

        def _scale(q, _e):
            vv = stv[slot, jj, pl.ds(q * L, L)]
            for i in range(L):
                v = vv[jnp.full((L,), i, jnp.int32)]
                e = q * L + i
                for f in range(D // L):
                    sl = pl.ds(f * L, L)
                    gbuf[b, e, sl] = gbuf[b, e, sl] * v
            return 0

        lax.fori_loop(0, CHUNK // L, _scale, 0)

        pltpu.async_copy(
            gbuf.at[b], acc.at[stg.at[slot, 1, jj]], ssem.at[b],
            add=True)
        return 0

    lax.fori_loop(0, NCHUNK, _chunk, 0)

    for p in range(NB):
        pltpu.make_async_copy(
            gbuf.at[p], acc.at[pl.ds(0, CHUNK)], ssem.at[p]).wait()
    plsc.subcore_barrier()

    def _dump(u, _):
        b = base_rows + u * UB
        pltpu.sync_copy(
            acc.at[pl.ds(b, UB)], partials.at[pl.ds(cbase + b, UB)])
        return 0

    lax.fori_loop(0, NUPD, _dump, 0)


def _mix_rows(dst, a_ref, h_ref):
    def _mix(i, _):
        for f in range(D // L):
            sl = pl.ds(f * L, L)
            dst[i, sl] = (1.0 - ALPHA) * (dst[i, sl] + a_ref[i, sl]) \
                + ALPHA * h_ref[i, sl]
        return 0

    lax.fori_loop(0, UB, _mix, 0)


def _body_upd(p01, x0_hbm, xfin, gbuf):
    c = lax.axis_index("c")
    s = lax.axis_index("s")
    base_rows = (c * NS + s) * NTF

    def _upd(u, _):
        b = base_rows + u * UB
        pltpu.sync_copy(p01.at[pl.ds(b, UB)], gbuf.at[0])
        pltpu.sync_copy(p01.at[pl.ds(NP2 + b, UB)], gbuf.at[1])
        pltpu.sync_copy(x0_hbm.at[pl.ds(b, UB)], gbuf.at[2])
        _mix_rows(gbuf.at[0], gbuf.at[1], gbuf.at[2])
        pltpu.sync_copy(gbuf.at[0], xfin.at[pl.ds(b, UB)])
        return 0

    lax.fori_loop(0, NUPF, _upd, 0)


@jax.jit
def kernel(x, adj_indices, adj_values):
    row = adj_indices[0].astype(jnp.int32)
    col = adj_indices[1].astype(jnp.int32)
    val = adj_values.astype(jnp.float32)

    pad = E_PAD - E
    row = jnp.concatenate([row, jnp.zeros((pad,), jnp.int32)])
    col = jnp.concatenate([col, jnp.zeros((pad,), jnp.int32)])
    val = jnp.concatenate([val, jnp.zeros((pad,), jnp.float32)])

    eidx = jnp.stack([
        col.reshape(NC, NS, NG, SB, CHUNK),
        row.reshape(NC, NS, NG, SB, CHUNK),
    ], axis=3)
    evals = val.reshape(NC, NS, NG, SB, CHUNK)

    x0 = jnp.pad(x, ((0, NP2 - N), (0, 0)))
    zeros = jnp.zeros((NT, D), jnp.float32)

    mesh = plsc.VectorSubcoreMesh(
        core_axis_name="c", subcore_axis_name="s", num_cores=NC)
    scratch = [
        pltpu.VMEM((3, 2, SB, CHUNK), jnp.int32),
        pltpu.VMEM((3, SB, CHUNK), jnp.float32),
        pltpu.VMEM((NB, CHUNK, D), jnp.float32),
        pltpu.VMEM_SHARED((NP2, D), jnp.float32),
        pltpu.SemaphoreType.DMA((NB,)),
        pltpu.SemaphoreType.DMA((NB,)),
        pltpu.SemaphoreType.DMA,
    ]

    b1 = pl.kernel(
        _body_b1,
        out_type=jax.ShapeDtypeStruct((NC * NP2, D), jnp.float32),
        mesh=mesh, scratch_types=scratch)
    upd = pl.kernel(
        _body_upd,
        out_type=jax.ShapeDtypeStruct((NP2, D), jnp.float32),
        mesh=mesh,
        scratch_types=[pltpu.VMEM((3, UB, D), jnp.float32)])

    parts = b1(x0, eidx, evals, zeros)
    for _ in range(K - 1):
        parts = b1(upd(parts, x0), eidx, evals, zeros)
    xfin = upd(parts, x0)
    return xfin[:N]


# speedup vs baseline: 2.2353x; 1.1031x over previous
import functools

import jax
import jax.numpy as jnp
from jax import lax
from jax.experimental import pallas as pl
from jax.experimental.pallas import tpu as pltpu
from jax.experimental.pallas import tpu_sc as plsc

N = 10000
E = 320000
D = 128
K = 10
ALPHA = 0.1

NC = 2
NS = 16
L = 16

CHUNK = 80
NB = 4
LA = NB - 1
SB = 4
GRP = SB * CHUNK
EP_TILE = -(-E // (NC * NS * GRP)) * GRP
NG = EP_TILE // GRP
NCHUNK = EP_TILE // CHUNK
E_PAD = EP_TILE * NC * NS

NP2 = 10240
NT = NP2 // NS
UB = 64
NUPD = NT // UB
NTF = NP2 // (NC * NS)
NUPF = NTF // UB


def _body_b1(x, eidx, evals, zeros_hbm, partials,
             stg, stv, gbuf, acc, gsem, ssem, stsem):
    c = lax.axis_index("c")
    s = lax.axis_index("s")
    base_rows = s * NT
    cbase = c * NP2

    pltpu.sync_copy(zeros_hbm, acc.at[pl.ds(base_rows, NT)])
    plsc.subcore_barrier()

    pltpu.async_copy(eidx.at[c, s, 0], stg.at[0], stsem)
    pltpu.async_copy(evals.at[c, s, 0], stv.at[0], stsem)
    pltpu.make_async_copy(eidx.at[c, s, 0], stg.at[0], stsem).wait()
    pltpu.make_async_copy(evals.at[c, s, 0], stv.at[0], stsem).wait()
    pltpu.async_copy(eidx.at[c, s, 1], stg.at[1], stsem)
    pltpu.async_copy(evals.at[c, s, 1], stv.at[1], stsem)
    for p in range(LA):
        g0, j0 = p // SB, p % SB
        pltpu.async_copy(
            x.at[stg.at[g0, 0, j0]], gbuf.at[p], gsem.at[p])

    def _chunk(j, _c):
        b = lax.rem(j, NB)
        g = lax.div(j, SB)
        jj = lax.rem(j, SB)
        slot = lax.rem(g, 3)

        pltpu.make_async_copy(
            x.at[stg.at[slot, 0, jj]], gbuf.at[b], gsem.at[b]).wait()

        @pl.when(j + LA < NCHUNK)
        def _pf():
            nb = lax.rem(j + LA, NB)

            @pl.when(j >= 1)
            def _ws():
                pltpu.make_async_copy(
                    gbuf.at[nb], acc.at[pl.ds(0, CHUNK)],
                    ssem.at[nb]).wait()

            @pl.when(lax.rem(j + LA, SB) == 0)
            def _cross():
                gx = lax.div(j + LA, SB)
                nslot = lax.rem(gx, 3)
                pltpu.make_async_copy(
                    eidx.at[c, s, gx], stg.at[nslot], stsem).wait()
                pltpu.make_async_copy(
                    evals.at[c, s, gx], stv.at[nslot], stsem).wait()

                @pl.when(gx + 1 < NG)
                def _st():
                    pltpu.async_copy(
                        eidx.at[c, s, gx + 1],
                        stg.at[lax.rem(gx + 1, 3)], stsem)
                    pltpu.async_copy(
                        evals.at[c, s, gx + 1],
                        stv.at[lax.rem(gx + 1, 3)], stsem)

            g1 = lax.div(j + LA, SB)
            jj1 = lax.rem(j + LA, SB)
            slot1 = lax.rem(g1, 3)
            pltpu.async_copy(
                x.at[stg.at[slot1, 0, jj1]], gbuf.at[nb], gsem.at[nb])

        for q in range(CHUNK // L):
            vv = stv[slot, jj, pl.ds(q * L, L)]
            for i in range(L):
                v = vv[jnp.full((L,), i, jnp.int32)]
                e = q * L + i
                for f in range(D // L):
                    sl = pl.ds(f * L, L)
                    gbuf[b, e, sl] = gbuf[b, e, sl] * v

        pltpu.async_copy(
            gbuf.at[b], acc.at[stg.at[slot, 1, jj]], ssem.at[b],
            add=True)
        return 0

    lax.fori_loop(0, NCHUNK, _chunk, 0)

    for p in range(NB):
        pltpu.make_async_copy(
            gbuf.at[p], acc.at[pl.ds(0, CHUNK)], ssem.at[p]).wait()
    plsc.subcore_barrier()

    def _dump(u, _):
        b = base_rows + u * UB
        pltpu.sync_copy(
            acc.at[pl.ds(b, UB)], partials.at[pl.ds(cbase + b, UB)])
        return 0

    lax.fori_loop(0, NUPD, _dump, 0)


def _mix_rows(dst, a_ref, h_ref):
    def _mix(i, _):
        for f in range(D // L):
            sl = pl.ds(f * L, L)
            dst[i, sl] = (1.0 - ALPHA) * (dst[i, sl] + a_ref[i, sl]) \
                + ALPHA * h_ref[i, sl]
        return 0

    lax.fori_loop(0, UB, _mix, 0)


def _body_upd(p01, x0_hbm, xfin, gbuf):
    c = lax.axis_index("c")
    s = lax.axis_index("s")
    base_rows = (c * NS + s) * NTF

    def _upd(u, _):
        b = base_rows + u * UB
        pltpu.sync_copy(p01.at[pl.ds(b, UB)], gbuf.at[0])
        pltpu.sync_copy(p01.at[pl.ds(NP2 + b, UB)], gbuf.at[1])
        pltpu.sync_copy(x0_hbm.at[pl.ds(b, UB)], gbuf.at[2])
        _mix_rows(gbuf.at[0], gbuf.at[1], gbuf.at[2])
        pltpu.sync_copy(gbuf.at[0], xfin.at[pl.ds(b, UB)])
        return 0

    lax.fori_loop(0, NUPF, _upd, 0)


@jax.jit
def kernel(x, adj_indices, adj_values):
    row = adj_indices[0].astype(jnp.int32)
    col = adj_indices[1].astype(jnp.int32)
    val = adj_values.astype(jnp.float32)

    pad = E_PAD - E
    row = jnp.concatenate([row, jnp.zeros((pad,), jnp.int32)])
    col = jnp.concatenate([col, jnp.zeros((pad,), jnp.int32)])
    val = jnp.concatenate([val, jnp.zeros((pad,), jnp.float32)])

    eidx = jnp.stack([
        col.reshape(NC, NS, NG, SB, CHUNK),
        row.reshape(NC, NS, NG, SB, CHUNK),
    ], axis=3)
    evals = val.reshape(NC, NS, NG, SB, CHUNK)

    x0 = jnp.pad(x, ((0, NP2 - N), (0, 0)))
    zeros = jnp.zeros((NT, D), jnp.float32)

    mesh = plsc.VectorSubcoreMesh(
        core_axis_name="c", subcore_axis_name="s", num_cores=NC)
    scratch = [
        pltpu.VMEM((3, 2, SB, CHUNK), jnp.int32),
        pltpu.VMEM((3, SB, CHUNK), jnp.float32),
        pltpu.VMEM((NB, CHUNK, D), jnp.float32),
        pltpu.VMEM_SHARED((NP2, D), jnp.float32),
        pltpu.SemaphoreType.DMA((NB,)),
        pltpu.SemaphoreType.DMA((NB,)),
        pltpu.SemaphoreType.DMA,
    ]

    b1 = pl.kernel(
        _body_b1,
        out_type=jax.ShapeDtypeStruct((NC * NP2, D), jnp.float32),
        mesh=mesh, scratch_types=scratch)
    upd = pl.kernel(
        _body_upd,
        out_type=jax.ShapeDtypeStruct((NP2, D), jnp.float32),
        mesh=mesh,
        scratch_types=[pltpu.VMEM((3, UB, D), jnp.float32)])

    parts = b1(x0, eidx, evals, zeros)
    for _ in range(K - 1):
        parts = b1(upd(parts, x0), eidx, evals, zeros)
    xfin = upd(parts, x0)
    return xfin[:N]
